# baseline (speedup 1.0000x reference)
import functools

import jax
import jax.numpy as jnp
from jax import lax
from jax.experimental import pallas as pl
from jax.experimental.pallas import tpu as pltpu
from jax.experimental.pallas import tpu_sc as plsc

DIM = 64
N = 4096 * 200
NC = 2
NS = 16
NW = NC * NS
PER_W = N // NW
C = 640
KSPLIT = 8
NCH = PER_W // C
NCH2 = NCH // 2

_mesh = plsc.VectorSubcoreMesh(core_axis_name="c", subcore_axis_name="s")


@functools.partial(
    pl.kernel,
    mesh=_mesh,
    out_type=jax.ShapeDtypeStruct((N, DIM), jnp.float32),
    scratch_types=[
        pltpu.VMEM((PER_W,), jnp.int32),
        pltpu.VMEM((C, DIM), jnp.float32),
        pltpu.VMEM((C, DIM), jnp.float32),
        pltpu.SemaphoreType.DMA,
        pltpu.SemaphoreType.DMA,
        pltpu.SemaphoreType.DMA,
    ],
    compiler_params=pltpu.CompilerParams(use_tc_tiling_on_sc=False),
)
def _emb_lookup(idx_hbm, table_hbm, out_hbm, idx_all, rows_v0, rows_v1,
                gsem, ssem0, ssem1):
    wid = lax.axis_index("s") * NC + lax.axis_index("c")
    base = wid * PER_W
    pltpu.sync_copy(idx_hbm.at[pl.ds(pl.multiple_of(base, 8), PER_W)], idx_all)

    S = C // KSPLIT

    def gather_chunk(ioff, rows_buf):
        for k in range(KSPLIT):
            o = pl.multiple_of(ioff + k * S, 8)
            pltpu.async_copy(
                table_hbm.at[idx_all.at[pl.ds(o, S)]],
                rows_buf.at[pl.ds(k * S, S)], gsem)
        pltpu.make_async_copy(
            table_hbm.at[idx_all.at[pl.ds(ioff, C)]], rows_buf, gsem).wait()

    def body(i, carry):
        e = 2 * i
        eo = pl.multiple_of(e * C, 8)
        oo = pl.multiple_of((e + 1) * C, 8)
        ebase = pl.multiple_of(base + e * C, 8)
        obase = pl.multiple_of(base + (e + 1) * C, 8)

        @pl.when(i > 0)
        def _():
            pltpu.make_async_copy(rows_v0, out_hbm.at[pl.ds(0, C)], ssem0).wait()

        gather_chunk(eo, rows_v0)
        pltpu.async_copy(rows_v0, out_hbm.at[pl.ds(ebase, C)], ssem0)

        @pl.when(i > 0)
        def _():
            pltpu.make_async_copy(rows_v1, out_hbm.at[pl.ds(0, C)], ssem1).wait()

        gather_chunk(oo, rows_v1)
        pltpu.async_copy(rows_v1, out_hbm.at[pl.ds(obase, C)], ssem1)
        return carry

    lax.fori_loop(0, NCH2, body, 0)
    pltpu.make_async_copy(rows_v0, out_hbm.at[pl.ds(0, C)], ssem0).wait()
    pltpu.make_async_copy(rows_v1, out_hbm.at[pl.ds(0, C)], ssem1).wait()


def kernel(token_ids, table):
    flat = token_ids.reshape(-1).astype(jnp.int32)
    out = _emb_lookup(flat, table)
    return out.reshape(token_ids.shape + (DIM,))
